# baseline (speedup 1.0000x reference)
import functools

import jax
import jax.numpy as jnp
from jax import lax
from jax.experimental import pallas as pl
from jax.experimental.pallas import tpu as pltpu
from jax.experimental.pallas import tpu_sc as plsc

N_NODES = 10000
HIDDEN = 40
OUT_DIM = 8
W = 48
L = 16
NC = 2
NS = 16
NW = NC * NS
CHUNK = 128
ROWS_PT = 632
ACC_ROWS = NS * ROWS_PT
ALPHA = 0.5
ALPHA_FCNN = 0.5
HI = jax.lax.Precision.HIGHEST


def _node_stage(x_ref, w1_ref, w2_ref, wd_ref, ws_ref, h_ref, hd_ref, hs_ref):
    h1 = jnp.maximum(jnp.dot(x_ref[...], w1_ref[...], precision=HI,
                             preferred_element_type=jnp.float32), 0.0)
    h = jnp.dot(h1, w2_ref[...], precision=HI, preferred_element_type=jnp.float32)
    h_ref[...] = h
    hd_ref[...] = jnp.dot(h, wd_ref[...], precision=HI, preferred_element_type=jnp.float32)
    hs_ref[...] = jnp.dot(h, ws_ref[...], precision=HI, preferred_element_type=jnp.float32)


def _edge_stage(ea_ref, w1_ref, w2_ref, wc_ref, brow_ref, ep_ref):
    t = jnp.maximum(jnp.dot(ea_ref[...], w1_ref[...], precision=HI,
                            preferred_element_type=jnp.float32), 0.0)
    e = jnp.dot(t, w2_ref[...], precision=HI, preferred_element_type=jnp.float32)
    ep_ref[...] = jnp.dot(e, wc_ref[...], precision=HI,
                          preferred_element_type=jnp.float32) + brow_ref[...]


def _out_stage(h_ref, p0_ref, p1_ref, xfc_ref, rpad_ref, o1h_ref, o1a_ref,
               ob1_ref, ow2_ref, ob2_ref, dw1_ref, dw2_ref, ln_ref, out_ref):
    p = p0_ref[...] + p1_ref[...]
    aggr = jnp.dot(p, rpad_ref[...], precision=HI, preferred_element_type=jnp.float32)
    h = h_ref[...]
    t = jnp.maximum(
        jnp.dot(h, o1h_ref[...], precision=HI, preferred_element_type=jnp.float32)
        + jnp.dot(aggr, o1a_ref[...], precision=HI, preferred_element_type=jnp.float32)
        + ob1_ref[...], 0.0)
    dx = jnp.dot(t, ow2_ref[...], precision=HI, preferred_element_type=jnp.float32) + ob2_ref[...]
    h2 = ALPHA * h + (1.0 - ALPHA) * dx
    d2 = jnp.dot(jnp.maximum(jnp.dot(h2, dw1_ref[...], precision=HI,
                                     preferred_element_type=jnp.float32), 0.0),
                 dw2_ref[...], precision=HI, preferred_element_type=jnp.float32)
    out_ref[...] = (ALPHA_FCNN * xfc_ref[...] + (1.0 - ALPHA_FCNN) * d2) * ln_ref[...]


def _make_sc_edge(n_edges, n_chunks_pw):
    mesh = plsc.VectorSubcoreMesh(
        core_axis_name="c", subcore_axis_name="s", num_cores=NC, num_subcores=NS)

    @functools.partial(
        pl.kernel,
        mesh=mesh,
        compiler_params=pltpu.CompilerParams(use_tc_tiling_on_sc=False),
        out_type=jax.ShapeDtypeStruct((NC, ACC_ROWS, W), jnp.float32),
        scratch_types=[
            pltpu.VMEM((CHUNK,), jnp.int32),
            pltpu.VMEM((CHUNK,), jnp.int32),
            pltpu.VMEM((CHUNK, W), jnp.float32),
            pltpu.VMEM((CHUNK, W), jnp.float32),
            pltpu.VMEM((CHUNK, W), jnp.float32),
            pltpu.VMEM_SHARED((ACC_ROWS, W), jnp.float32),
            pltpu.SemaphoreType.DMA,
            pltpu.SemaphoreType.DMA,
        ],
    )
    def sc_edge(dst_hbm, src_hbm, ep_hbm, hd_hbm, hs_hbm, zero_hbm, out_hbm,
                dix, six, av, bv, cv, acc, sem_a, sem_b):
        cid = lax.axis_index("c")
        sid = lax.axis_index("s")
        pltpu.sync_copy(zero_hbm, acc.at[pl.ds(sid * ROWS_PT, ROWS_PT)])
        plsc.subcore_barrier()
        base = (cid * NS + sid) * (n_chunks_pw * CHUNK)

        def body(g, carry):
            eb_raw = base + g * CHUNK
            valid = eb_raw <= n_edges - CHUNK
            eb = jnp.minimum(eb_raw, n_edges - CHUNK)
            gate = jnp.where(valid, 1.0, 0.0).astype(jnp.float32)
            pltpu.sync_copy(dst_hbm.at[pl.ds(eb, CHUNK)], dix)
            pltpu.sync_copy(src_hbm.at[pl.ds(eb, CHUNK)], six)
            cpa = pltpu.async_copy(hd_hbm.at[dix], av, sem_a)
            cpb = pltpu.async_copy(hs_hbm.at[six], bv, sem_b)
            pltpu.sync_copy(ep_hbm.at[pl.ds(eb, CHUNK)], cv)
            cpa.wait()
            cpb.wait()

            def inner(i, c2):
                for j in range(W // L):
                    sl = pl.ds(j * L, L)
                    cv[i, sl] = jnp.maximum(av[i, sl] + bv[i, sl] + cv[i, sl], 0.0) * gate
                return c2

            lax.fori_loop(0, CHUNK, inner, 0, unroll=2)
            pltpu.sync_copy(cv, acc.at[dix], add=True)
            return carry

        lax.fori_loop(0, n_chunks_pw, body, 0)
        plsc.subcore_barrier()
        pltpu.sync_copy(acc.at[pl.ds(sid * ROWS_PT, ROWS_PT)],
                        out_hbm.at[cid, pl.ds(sid * ROWS_PT, ROWS_PT)])

    return sc_edge


@jax.jit
def kernel(x, edge_index, edge_attr, ne_w1, ne_w2, ee_w1, ee_w2, rel_w1,
           rel_b1, rel_w2, rel_b2, obj_w1, obj_b1, obj_w2, obj_b2, de_w1,
           de_w2, latent_norm):
    f32 = jnp.float32
    n = x.shape[0]
    e_cnt = edge_attr.shape[0]
    assert e_cnt % CHUNK == 0

    def pad48(w):
        return jnp.concatenate([w, jnp.zeros((w.shape[0], W - HIDDEN), w.dtype)], axis=1)

    wd = pad48(rel_w1[0:HIDDEN])
    ws = pad48(rel_w1[HIDDEN:2 * HIDDEN])
    wc = pad48(rel_w1[2 * HIDDEN:3 * HIDDEN])
    brow = jnp.concatenate(
        [rel_b1, jnp.ones((1,), f32), jnp.zeros((W - HIDDEN - 1,), f32)]).reshape(1, W)
    rpad = jnp.concatenate(
        [rel_w2, rel_b2.reshape(1, HIDDEN), jnp.zeros((W - HIDDEN - 1, HIDDEN), f32)], axis=0)

    h, hd, hs = pl.pallas_call(
        _node_stage,
        out_shape=[
            jax.ShapeDtypeStruct((n, HIDDEN), f32),
            jax.ShapeDtypeStruct((n, W), f32),
            jax.ShapeDtypeStruct((n, W), f32),
        ],
    )(x, ne_w1, ne_w2, wd, ws)

    eb = 4000
    ep = pl.pallas_call(
        _edge_stage,
        grid=(e_cnt // eb,),
        in_specs=[
            pl.BlockSpec((eb, edge_attr.shape[1]), lambda i: (i, 0)),
            pl.BlockSpec(ee_w1.shape, lambda i: (0, 0)),
            pl.BlockSpec(ee_w2.shape, lambda i: (0, 0)),
            pl.BlockSpec((HIDDEN, W), lambda i: (0, 0)),
            pl.BlockSpec((1, W), lambda i: (0, 0)),
        ],
        out_specs=pl.BlockSpec((eb, W), lambda i: (i, 0)),
        out_shape=jax.ShapeDtypeStruct((e_cnt, W), f32),
    )(edge_attr, ee_w1, ee_w2, wc, brow)

    n_chunks = e_cnt // CHUNK
    n_chunks_pw = (n_chunks + NW - 1) // NW
    zeros_tile = jnp.zeros((ROWS_PT, W), f32)

    parts = _make_sc_edge(e_cnt, n_chunks_pw)(
        edge_index[1], edge_index[0], ep, hd, hs, zeros_tile)

    out = pl.pallas_call(
        _out_stage,
        out_shape=jax.ShapeDtypeStruct((n, OUT_DIM), f32),
    )(h, parts[0, :n], parts[1, :n], x[:, :OUT_DIM], rpad,
      obj_w1[:HIDDEN], obj_w1[HIDDEN:], obj_b1.reshape(1, HIDDEN),
      obj_w2, obj_b2.reshape(1, HIDDEN), de_w1, de_w2, latent_norm.reshape(1, 1))
    return out


# speedup vs baseline: 2.0855x; 2.0855x over previous
import functools

import jax
import jax.numpy as jnp
from jax import lax
from jax.experimental import pallas as pl
from jax.experimental.pallas import tpu as pltpu
from jax.experimental.pallas import tpu_sc as plsc

N_NODES = 10000
HIDDEN = 40
OUT_DIM = 8
W = 48
L = 16
NC = 2
NS = 16
NW = NC * NS
CHUNK = 128
ROWS_PT = 632
ACC_ROWS = NS * ROWS_PT
ALPHA = 0.5
ALPHA_FCNN = 0.5


def _node_stage(x_ref, w1_ref, w2_ref, wd_ref, ws_ref, h_ref, hd_ref, hs_ref):
    h1 = jnp.maximum(jnp.dot(x_ref[...], w1_ref[...], preferred_element_type=jnp.float32), 0.0)
    h = jnp.dot(h1, w2_ref[...], preferred_element_type=jnp.float32)
    h_ref[...] = h
    hd_ref[...] = jnp.dot(h, wd_ref[...], preferred_element_type=jnp.float32)
    hs_ref[...] = jnp.dot(h, ws_ref[...], preferred_element_type=jnp.float32)


def _edge_stage(ea_ref, w1_ref, w2_ref, wc_ref, brow_ref, ep_ref):
    t = jnp.maximum(jnp.dot(ea_ref[...], w1_ref[...], preferred_element_type=jnp.float32), 0.0)
    e = jnp.dot(t, w2_ref[...], preferred_element_type=jnp.float32)
    ep_ref[...] = jnp.dot(e, wc_ref[...], preferred_element_type=jnp.float32) + brow_ref[...]


def _out_stage(h_ref, p0_ref, p1_ref, xfc_ref, rpad_ref, o1h_ref, o1a_ref,
               ob1_ref, ow2_ref, ob2_ref, dw1_ref, dw2_ref, ln_ref, out_ref):
    p = p0_ref[...] + p1_ref[...]
    aggr = jnp.dot(p, rpad_ref[...], preferred_element_type=jnp.float32)
    h = h_ref[...]
    t = jnp.maximum(
        jnp.dot(h, o1h_ref[...], preferred_element_type=jnp.float32)
        + jnp.dot(aggr, o1a_ref[...], preferred_element_type=jnp.float32)
        + ob1_ref[...], 0.0)
    dx = jnp.dot(t, ow2_ref[...], preferred_element_type=jnp.float32) + ob2_ref[...]
    h2 = ALPHA * h + (1.0 - ALPHA) * dx
    d2 = jnp.dot(jnp.maximum(jnp.dot(h2, dw1_ref[...], preferred_element_type=jnp.float32), 0.0),
                 dw2_ref[...], preferred_element_type=jnp.float32)
    out_ref[...] = (ALPHA_FCNN * xfc_ref[...] + (1.0 - ALPHA_FCNN) * d2) * ln_ref[...]


def _make_sc_edge(n_edges, n_chunks_pw):
    mesh = plsc.VectorSubcoreMesh(
        core_axis_name="c", subcore_axis_name="s", num_cores=NC, num_subcores=NS)

    @functools.partial(
        pl.kernel,
        mesh=mesh,
        compiler_params=pltpu.CompilerParams(use_tc_tiling_on_sc=False),
        out_type=jax.ShapeDtypeStruct((NC, ACC_ROWS, W), jnp.float32),
        scratch_types=[
            pltpu.VMEM((CHUNK,), jnp.int32),
            pltpu.VMEM((CHUNK,), jnp.int32),
            pltpu.VMEM((CHUNK, W), jnp.float32),
            pltpu.VMEM((CHUNK, W), jnp.float32),
            pltpu.VMEM((CHUNK, W), jnp.float32),
            pltpu.VMEM_SHARED((ACC_ROWS, W), jnp.float32),
            pltpu.SemaphoreType.DMA,
            pltpu.SemaphoreType.DMA,
        ],
    )
    def sc_edge(dst_hbm, src_hbm, ep_hbm, hd_hbm, hs_hbm, zero_hbm, out_hbm,
                dix, six, av, bv, cv, acc, sem_a, sem_b):
        cid = lax.axis_index("c")
        sid = lax.axis_index("s")
        pltpu.sync_copy(zero_hbm, acc.at[pl.ds(sid * ROWS_PT, ROWS_PT)])
        plsc.subcore_barrier()
        base = (cid * NS + sid) * (n_chunks_pw * CHUNK)

        def body(g, carry):
            eb_raw = base + g * CHUNK
            valid = eb_raw <= n_edges - CHUNK
            eb = jnp.minimum(eb_raw, n_edges - CHUNK)
            gate = jnp.where(valid, 1.0, 0.0).astype(jnp.float32)
            pltpu.sync_copy(dst_hbm.at[pl.ds(eb, CHUNK)], dix)
            pltpu.sync_copy(src_hbm.at[pl.ds(eb, CHUNK)], six)
            cpa = pltpu.async_copy(hd_hbm.at[dix], av, sem_a)
            cpb = pltpu.async_copy(hs_hbm.at[six], bv, sem_b)
            pltpu.sync_copy(ep_hbm.at[pl.ds(eb, CHUNK)], cv)
            cpa.wait()
            cpb.wait()

            def inner(i, c2):
                for j in range(W // L):
                    sl = pl.ds(j * L, L)
                    cv[i, sl] = jnp.maximum(av[i, sl] + bv[i, sl] + cv[i, sl], 0.0) * gate
                return c2

            lax.fori_loop(0, CHUNK, inner, 0)
            pltpu.sync_copy(cv, acc.at[dix], add=True)
            return carry

        lax.fori_loop(0, n_chunks_pw, body, 0)
        plsc.subcore_barrier()
        pltpu.sync_copy(acc.at[pl.ds(sid * ROWS_PT, ROWS_PT)],
                        out_hbm.at[cid, pl.ds(sid * ROWS_PT, ROWS_PT)])

    return sc_edge


@jax.jit
def kernel(x, edge_index, edge_attr, ne_w1, ne_w2, ee_w1, ee_w2, rel_w1,
           rel_b1, rel_w2, rel_b2, obj_w1, obj_b1, obj_w2, obj_b2, de_w1,
           de_w2, latent_norm):
    f32 = jnp.float32
    n = x.shape[0]
    e_cnt = edge_attr.shape[0]
    assert e_cnt % CHUNK == 0

    def pad48(w):
        return jnp.concatenate([w, jnp.zeros((w.shape[0], W - HIDDEN), w.dtype)], axis=1)

    wd = pad48(rel_w1[0:HIDDEN])
    ws = pad48(rel_w1[HIDDEN:2 * HIDDEN])
    wc = pad48(rel_w1[2 * HIDDEN:3 * HIDDEN])
    brow = jnp.concatenate(
        [rel_b1, jnp.ones((1,), f32), jnp.zeros((W - HIDDEN - 1,), f32)]).reshape(1, W)
    rpad = jnp.concatenate(
        [rel_w2, rel_b2.reshape(1, HIDDEN), jnp.zeros((W - HIDDEN - 1, HIDDEN), f32)], axis=0)

    h, hd, hs = pl.pallas_call(
        _node_stage,
        out_shape=[
            jax.ShapeDtypeStruct((n, HIDDEN), f32),
            jax.ShapeDtypeStruct((n, W), f32),
            jax.ShapeDtypeStruct((n, W), f32),
        ],
    )(x, ne_w1, ne_w2, wd, ws)

    eb = 20000
    ep = pl.pallas_call(
        _edge_stage,
        grid=(e_cnt // eb,),
        in_specs=[
            pl.BlockSpec((eb, edge_attr.shape[1]), lambda i: (i, 0)),
            pl.BlockSpec(ee_w1.shape, lambda i: (0, 0)),
            pl.BlockSpec(ee_w2.shape, lambda i: (0, 0)),
            pl.BlockSpec((HIDDEN, W), lambda i: (0, 0)),
            pl.BlockSpec((1, W), lambda i: (0, 0)),
        ],
        out_specs=pl.BlockSpec((eb, W), lambda i: (i, 0)),
        out_shape=jax.ShapeDtypeStruct((e_cnt, W), f32),
    )(edge_attr, ee_w1, ee_w2, wc, brow)

    n_chunks = e_cnt // CHUNK
    n_chunks_pw = (n_chunks + NW - 1) // NW
    zeros_tile = jnp.zeros((ROWS_PT, W), f32)

    parts = _make_sc_edge(e_cnt, n_chunks_pw)(
        edge_index[1], edge_index[0], ep, hd, hs, zeros_tile)

    out = pl.pallas_call(
        _out_stage,
        out_shape=jax.ShapeDtypeStruct((n, OUT_DIM), f32),
    )(h, parts[0, :n], parts[1, :n], x[:, :OUT_DIM], rpad,
      obj_w1[:HIDDEN], obj_w1[HIDDEN:], obj_b1.reshape(1, HIDDEN),
      obj_w2, obj_b2.reshape(1, HIDDEN), de_w1, de_w2, latent_norm.reshape(1, 1))
    return out


# speedup vs baseline: 2.3543x; 1.1289x over previous
import functools

import jax
import jax.numpy as jnp
from jax import lax
from jax.experimental import pallas as pl
from jax.experimental.pallas import tpu as pltpu
from jax.experimental.pallas import tpu_sc as plsc

N_NODES = 10000
HIDDEN = 40
OUT_DIM = 8
W = 48
L = 16
NC = 2
NS = 16
NW = NC * NS
CHUNK = 128
ROWS_PT = 632
ACC_ROWS = NS * ROWS_PT
ALPHA = 0.5
ALPHA_FCNN = 0.5


def _node_stage(x_ref, w1_ref, w2_ref, wd_ref, ws_ref, h_ref, hd_ref, hs_ref):
    h1 = jnp.maximum(jnp.dot(x_ref[...], w1_ref[...], preferred_element_type=jnp.float32), 0.0)
    h = jnp.dot(h1, w2_ref[...], preferred_element_type=jnp.float32)
    h_ref[...] = h
    hd_ref[...] = jnp.dot(h, wd_ref[...], preferred_element_type=jnp.float32)
    hs_ref[...] = jnp.dot(h, ws_ref[...], preferred_element_type=jnp.float32)


def _edge_stage(ea_ref, w1_ref, w2_ref, wc_ref, brow_ref, ep_ref):
    t = jnp.maximum(jnp.dot(ea_ref[...], w1_ref[...], preferred_element_type=jnp.float32), 0.0)
    e = jnp.dot(t, w2_ref[...], preferred_element_type=jnp.float32)
    ep_ref[...] = jnp.dot(e, wc_ref[...], preferred_element_type=jnp.float32) + brow_ref[...]


def _out_stage(h_ref, p0_ref, p1_ref, xfc_ref, rpad_ref, o1h_ref, o1a_ref,
               ob1_ref, ow2_ref, ob2_ref, dw1_ref, dw2_ref, ln_ref, out_ref):
    p = p0_ref[...] + p1_ref[...]
    aggr = jnp.dot(p, rpad_ref[...], preferred_element_type=jnp.float32)
    h = h_ref[...]
    t = jnp.maximum(
        jnp.dot(h, o1h_ref[...], preferred_element_type=jnp.float32)
        + jnp.dot(aggr, o1a_ref[...], preferred_element_type=jnp.float32)
        + ob1_ref[...], 0.0)
    dx = jnp.dot(t, ow2_ref[...], preferred_element_type=jnp.float32) + ob2_ref[...]
    h2 = ALPHA * h + (1.0 - ALPHA) * dx
    d2 = jnp.dot(jnp.maximum(jnp.dot(h2, dw1_ref[...], preferred_element_type=jnp.float32), 0.0),
                 dw2_ref[...], preferred_element_type=jnp.float32)
    out_ref[...] = (ALPHA_FCNN * xfc_ref[...] + (1.0 - ALPHA_FCNN) * d2) * ln_ref[...]


def _make_sc_edge(n_edges, n_slots_pw):
    mesh = plsc.VectorSubcoreMesh(
        core_axis_name="c", subcore_axis_name="s", num_cores=NC, num_subcores=NS)

    @functools.partial(
        pl.kernel,
        mesh=mesh,
        compiler_params=pltpu.CompilerParams(use_tc_tiling_on_sc=False),
        out_type=jax.ShapeDtypeStruct((NC, ACC_ROWS, W), jnp.float32),
        scratch_types=[
            pltpu.VMEM((CHUNK,), jnp.int32), pltpu.VMEM((CHUNK,), jnp.int32),
            pltpu.VMEM((CHUNK,), jnp.int32), pltpu.VMEM((CHUNK,), jnp.int32),
            pltpu.VMEM((CHUNK, W), jnp.float32), pltpu.VMEM((CHUNK, W), jnp.float32),
            pltpu.VMEM((CHUNK, W), jnp.float32), pltpu.VMEM((CHUNK, W), jnp.float32),
            pltpu.VMEM((CHUNK, W), jnp.float32), pltpu.VMEM((CHUNK, W), jnp.float32),
            pltpu.VMEM_SHARED((ACC_ROWS, W), jnp.float32),
            pltpu.SemaphoreType.DMA, pltpu.SemaphoreType.DMA,
            pltpu.SemaphoreType.DMA, pltpu.SemaphoreType.DMA,
            pltpu.SemaphoreType.DMA, pltpu.SemaphoreType.DMA,
        ],
    )
    def sc_edge(dst_hbm, src_hbm, ep_hbm, hd_hbm, hs_hbm, zero_hbm, out_hbm,
                dix0, six0, dix1, six1, av0, av1, bv0, bv1, cv0, cv1, acc,
                sa0, sa1, sb0, sb1, se0, se1):
        dix = (dix0, dix1)
        six = (six0, six1)
        av = (av0, av1)
        bv = (bv0, bv1)
        cv = (cv0, cv1)
        sa = (sa0, sa1)
        sb = (sb0, sb1)
        se = (se0, se1)
        cid = lax.axis_index("c")
        sid = lax.axis_index("s")
        pltpu.sync_copy(zero_hbm, acc.at[pl.ds(sid * ROWS_PT, ROWS_PT)])
        plsc.subcore_barrier()
        base = (cid * NS + sid) * (n_slots_pw * CHUNK)
        last = n_edges - CHUNK

        def issue(g, b):
            eb = jnp.minimum(base + g * CHUNK, last)
            pltpu.sync_copy(dst_hbm.at[pl.ds(eb, CHUNK)], dix[b])
            pltpu.sync_copy(src_hbm.at[pl.ds(eb, CHUNK)], six[b])
            cpe = pltpu.async_copy(ep_hbm.at[pl.ds(eb, CHUNK)], cv[b], se[b])
            cpa = pltpu.async_copy(hd_hbm.at[dix[b]], av[b], sa[b])
            cpb = pltpu.async_copy(hs_hbm.at[six[b]], bv[b], sb[b])
            return (cpe, cpa, cpb)

        def drain(g, b, handles):
            gate = jnp.where(base + g * CHUNK <= last, 1.0, 0.0).astype(jnp.float32)
            for hnd in handles:
                hnd.wait()

            def inner(i, c2):
                for j in range(W // L):
                    sl = pl.ds(j * L, L)
                    cv[b][i, sl] = jnp.maximum(
                        av[b][i, sl] + bv[b][i, sl] + cv[b][i, sl], 0.0) * gate
                return c2

            lax.fori_loop(0, CHUNK, inner, 0)
            pltpu.sync_copy(cv[b], acc.at[dix[b]], add=True)

        def body(k, carry):
            g0 = 2 * k
            h0 = issue(g0, 0)
            h1 = issue(g0 + 1, 1)
            drain(g0, 0, h0)
            drain(g0 + 1, 1, h1)
            return carry

        lax.fori_loop(0, n_slots_pw // 2, body, 0)
        plsc.subcore_barrier()
        pltpu.sync_copy(acc.at[pl.ds(sid * ROWS_PT, ROWS_PT)],
                        out_hbm.at[cid, pl.ds(sid * ROWS_PT, ROWS_PT)])

    return sc_edge


@jax.jit
def kernel(x, edge_index, edge_attr, ne_w1, ne_w2, ee_w1, ee_w2, rel_w1,
           rel_b1, rel_w2, rel_b2, obj_w1, obj_b1, obj_w2, obj_b2, de_w1,
           de_w2, latent_norm):
    f32 = jnp.float32
    n = x.shape[0]
    e_cnt = edge_attr.shape[0]
    assert e_cnt % CHUNK == 0

    def pad48(w):
        return jnp.concatenate([w, jnp.zeros((w.shape[0], W - HIDDEN), w.dtype)], axis=1)

    wd = pad48(rel_w1[0:HIDDEN])
    ws = pad48(rel_w1[HIDDEN:2 * HIDDEN])
    wc = pad48(rel_w1[2 * HIDDEN:3 * HIDDEN])
    brow = jnp.concatenate(
        [rel_b1, jnp.ones((1,), f32), jnp.zeros((W - HIDDEN - 1,), f32)]).reshape(1, W)
    rpad = jnp.concatenate(
        [rel_w2, rel_b2.reshape(1, HIDDEN), jnp.zeros((W - HIDDEN - 1, HIDDEN), f32)], axis=0)

    h, hd, hs = pl.pallas_call(
        _node_stage,
        out_shape=[
            jax.ShapeDtypeStruct((n, HIDDEN), f32),
            jax.ShapeDtypeStruct((n, W), f32),
            jax.ShapeDtypeStruct((n, W), f32),
        ],
    )(x, ne_w1, ne_w2, wd, ws)

    eb = 20000
    ep = pl.pallas_call(
        _edge_stage,
        grid=(e_cnt // eb,),
        in_specs=[
            pl.BlockSpec((eb, edge_attr.shape[1]), lambda i: (i, 0)),
            pl.BlockSpec(ee_w1.shape, lambda i: (0, 0)),
            pl.BlockSpec(ee_w2.shape, lambda i: (0, 0)),
            pl.BlockSpec((HIDDEN, W), lambda i: (0, 0)),
            pl.BlockSpec((1, W), lambda i: (0, 0)),
        ],
        out_specs=pl.BlockSpec((eb, W), lambda i: (i, 0)),
        out_shape=jax.ShapeDtypeStruct((e_cnt, W), f32),
    )(edge_attr, ee_w1, ee_w2, wc, brow)

    n_chunks = e_cnt // CHUNK
    n_chunks_pw = (n_chunks + NW - 1) // NW
    n_chunks_pw += n_chunks_pw % 2
    zeros_tile = jnp.zeros((ROWS_PT, W), f32)

    parts = _make_sc_edge(e_cnt, n_chunks_pw)(
        edge_index[1], edge_index[0], ep, hd, hs, zeros_tile)

    out = pl.pallas_call(
        _out_stage,
        out_shape=jax.ShapeDtypeStruct((n, OUT_DIM), f32),
    )(h, parts[0, :n], parts[1, :n], x[:, :OUT_DIM], rpad,
      obj_w1[:HIDDEN], obj_w1[HIDDEN:], obj_b1.reshape(1, HIDDEN),
      obj_w2, obj_b2.reshape(1, HIDDEN), de_w1, de_w2, latent_norm.reshape(1, 1))
    return out


# speedup vs baseline: 2.7994x; 1.1890x over previous
import functools

import jax
import jax.numpy as jnp
from jax import lax
from jax.experimental import pallas as pl
from jax.experimental.pallas import tpu as pltpu
from jax.experimental.pallas import tpu_sc as plsc

N_NODES = 10000
HIDDEN = 40
OUT_DIM = 8
W = 48
L = 16
NC = 2
NS = 16
NW = NC * NS
CHUNK = 128
ROWS_PT = 632
ACC_ROWS = NS * ROWS_PT
ALPHA = 0.5
ALPHA_FCNN = 0.5


def _node_stage(x_ref, w1_ref, w2_ref, wd_ref, ws_ref, h_ref, hd_ref, hs_ref):
    h1 = jnp.maximum(jnp.dot(x_ref[...], w1_ref[...], preferred_element_type=jnp.float32), 0.0)
    h = jnp.dot(h1, w2_ref[...], preferred_element_type=jnp.float32)
    h_ref[...] = h
    hd_ref[...] = jnp.dot(h, wd_ref[...], preferred_element_type=jnp.float32)
    hs_ref[...] = jnp.dot(h, ws_ref[...], preferred_element_type=jnp.float32)


def _edge_stage(ea_ref, w1_ref, w2_ref, wc_ref, brow_ref, ep_ref):
    t = jnp.maximum(jnp.dot(ea_ref[...], w1_ref[...], preferred_element_type=jnp.float32), 0.0)
    e = jnp.dot(t, w2_ref[...], preferred_element_type=jnp.float32)
    ep_ref[...] = jnp.dot(e, wc_ref[...], preferred_element_type=jnp.float32) + brow_ref[...]


def _out_stage(h_ref, p0_ref, p1_ref, xfc_ref, rpad_ref, o1h_ref, o1a_ref,
               ob1_ref, ow2_ref, ob2_ref, dw1_ref, dw2_ref, ln_ref, out_ref):
    p = p0_ref[...] + p1_ref[...]
    aggr = jnp.dot(p, rpad_ref[...], preferred_element_type=jnp.float32)
    h = h_ref[...]
    t = jnp.maximum(
        jnp.dot(h, o1h_ref[...], preferred_element_type=jnp.float32)
        + jnp.dot(aggr, o1a_ref[...], preferred_element_type=jnp.float32)
        + ob1_ref[...], 0.0)
    dx = jnp.dot(t, ow2_ref[...], preferred_element_type=jnp.float32) + ob2_ref[...]
    h2 = ALPHA * h + (1.0 - ALPHA) * dx
    d2 = jnp.dot(jnp.maximum(jnp.dot(h2, dw1_ref[...], preferred_element_type=jnp.float32), 0.0),
                 dw2_ref[...], preferred_element_type=jnp.float32)
    out_ref[...] = (ALPHA_FCNN * xfc_ref[...] + (1.0 - ALPHA_FCNN) * d2) * ln_ref[...]


def _make_sc_edge(n_edges, n_slots_pw):
    mesh = plsc.VectorSubcoreMesh(
        core_axis_name="c", subcore_axis_name="s", num_cores=NC, num_subcores=NS)

    @functools.partial(
        pl.kernel,
        mesh=mesh,
        compiler_params=pltpu.CompilerParams(use_tc_tiling_on_sc=False),
        out_type=jax.ShapeDtypeStruct((NC, ACC_ROWS, W), jnp.float32),
        scratch_types=[
            pltpu.VMEM((CHUNK,), jnp.int32), pltpu.VMEM((CHUNK,), jnp.int32),
            pltpu.VMEM((CHUNK,), jnp.int32), pltpu.VMEM((CHUNK,), jnp.int32),
            pltpu.VMEM((CHUNK, W), jnp.float32), pltpu.VMEM((CHUNK, W), jnp.float32),
            pltpu.VMEM((CHUNK, W), jnp.float32), pltpu.VMEM((CHUNK, W), jnp.float32),
            pltpu.VMEM((CHUNK // 8, 8 * W), jnp.float32),
            pltpu.VMEM((CHUNK // 8, 8 * W), jnp.float32),
            pltpu.VMEM((CHUNK, W), jnp.float32), pltpu.VMEM((CHUNK, W), jnp.float32),
            pltpu.VMEM_SHARED((ACC_ROWS, W), jnp.float32),
            pltpu.SemaphoreType.DMA, pltpu.SemaphoreType.DMA,
            pltpu.SemaphoreType.DMA, pltpu.SemaphoreType.DMA,
            pltpu.SemaphoreType.DMA, pltpu.SemaphoreType.DMA,
        ],
    )
    def sc_edge(dst_hbm, src_hbm, ep8_hbm, hd_hbm, hs_hbm, zero_hbm, out_hbm,
                dix0, six0, dix1, six1, av0, av1, bv0, bv1, cv80, cv81,
                cv0, cv1, acc, sa0, sa1, sb0, sb1, se0, se1):
        dix = (dix0, dix1)
        six = (six0, six1)
        av = (av0, av1)
        bv = (bv0, bv1)
        cv8 = (cv80, cv81)
        cv = (cv0, cv1)
        sa = (sa0, sa1)
        sb = (sb0, sb1)
        se = (se0, se1)
        cid = lax.axis_index("c")
        sid = lax.axis_index("s")
        pltpu.sync_copy(zero_hbm, acc.at[pl.ds(sid * ROWS_PT, ROWS_PT)])
        plsc.subcore_barrier()
        base = (cid * NS + sid) * (n_slots_pw * CHUNK)
        last = n_edges - CHUNK

        def issue(g, b):
            eb = jnp.minimum(base + g * CHUNK, last)
            pltpu.sync_copy(dst_hbm.at[pl.ds(eb, CHUNK)], dix[b])
            pltpu.sync_copy(src_hbm.at[pl.ds(eb, CHUNK)], six[b])
            cpe = pltpu.async_copy(ep8_hbm.at[pl.ds(eb // 8, CHUNK // 8)], cv8[b], se[b])
            cpa = pltpu.async_copy(hd_hbm.at[dix[b]], av[b], sa[b])
            cpb = pltpu.async_copy(hs_hbm.at[six[b]], bv[b], sb[b])
            return (cpe, cpa, cpb)

        def drain(g, b, handles):
            gate = jnp.where(base + g * CHUNK <= last, 1.0, 0.0).astype(jnp.float32)
            for hnd in handles:
                hnd.wait()

            def inner(r, c2):
                for k in range(8):
                    i = r * 8 + k
                    for j in range(W // L):
                        cv[b][i, pl.ds(j * L, L)] = jnp.maximum(
                            av[b][i, pl.ds(j * L, L)] + bv[b][i, pl.ds(j * L, L)]
                            + cv8[b][r, pl.ds(k * W + j * L, L)], 0.0) * gate
                return c2

            lax.fori_loop(0, CHUNK // 8, inner, 0)
            pltpu.sync_copy(cv[b], acc.at[dix[b]], add=True)

        def body(k, carry):
            g0 = 2 * k
            h0 = issue(g0, 0)
            h1 = issue(g0 + 1, 1)
            drain(g0, 0, h0)
            drain(g0 + 1, 1, h1)
            return carry

        lax.fori_loop(0, n_slots_pw // 2, body, 0)
        plsc.subcore_barrier()
        pltpu.sync_copy(acc.at[pl.ds(sid * ROWS_PT, ROWS_PT)],
                        out_hbm.at[cid, pl.ds(sid * ROWS_PT, ROWS_PT)])

    return sc_edge


@jax.jit
def kernel(x, edge_index, edge_attr, ne_w1, ne_w2, ee_w1, ee_w2, rel_w1,
           rel_b1, rel_w2, rel_b2, obj_w1, obj_b1, obj_w2, obj_b2, de_w1,
           de_w2, latent_norm):
    f32 = jnp.float32
    n = x.shape[0]
    e_cnt = edge_attr.shape[0]
    assert e_cnt % CHUNK == 0

    def pad48(w):
        return jnp.concatenate([w, jnp.zeros((w.shape[0], W - HIDDEN), w.dtype)], axis=1)

    wd = pad48(rel_w1[0:HIDDEN])
    ws = pad48(rel_w1[HIDDEN:2 * HIDDEN])
    wc = pad48(rel_w1[2 * HIDDEN:3 * HIDDEN])
    brow = jnp.concatenate(
        [rel_b1, jnp.ones((1,), f32), jnp.zeros((W - HIDDEN - 1,), f32)]).reshape(1, W)

    def blockdiag8(w):
        a, b = w.shape
        out = jnp.zeros((8 * a, 8 * b), w.dtype)
        for k in range(8):
            out = lax.dynamic_update_slice(out, w, (k * a, k * b))
        return out

    w1_blk = blockdiag8(ee_w1)
    w2_blk = blockdiag8(ee_w2)
    wc_blk = blockdiag8(wc)
    brow_blk = jnp.tile(brow, (1, 8))
    rpad = jnp.concatenate(
        [rel_w2, rel_b2.reshape(1, HIDDEN), jnp.zeros((W - HIDDEN - 1, HIDDEN), f32)], axis=0)

    h, hd, hs = pl.pallas_call(
        _node_stage,
        out_shape=[
            jax.ShapeDtypeStruct((n, HIDDEN), f32),
            jax.ShapeDtypeStruct((n, W), f32),
            jax.ShapeDtypeStruct((n, W), f32),
        ],
    )(x, ne_w1, ne_w2, wd, ws)

    e8 = e_cnt // 8
    ea8 = edge_attr.reshape(e8, 8 * edge_attr.shape[1])
    eb8 = 2000
    ep8 = pl.pallas_call(
        _edge_stage,
        grid=(e8 // eb8,),
        in_specs=[
            pl.BlockSpec((eb8, ea8.shape[1]), lambda i: (i, 0)),
            pl.BlockSpec(w1_blk.shape, lambda i: (0, 0)),
            pl.BlockSpec(w2_blk.shape, lambda i: (0, 0)),
            pl.BlockSpec(wc_blk.shape, lambda i: (0, 0)),
            pl.BlockSpec((1, 8 * W), lambda i: (0, 0)),
        ],
        out_specs=pl.BlockSpec((eb8, 8 * W), lambda i: (i, 0)),
        out_shape=jax.ShapeDtypeStruct((e8, 8 * W), f32),
    )(ea8, w1_blk, w2_blk, wc_blk, brow_blk)

    n_chunks = e_cnt // CHUNK
    n_chunks_pw = (n_chunks + NW - 1) // NW
    n_chunks_pw += n_chunks_pw % 2
    zeros_tile = jnp.zeros((ROWS_PT, W), f32)

    parts = _make_sc_edge(e_cnt, n_chunks_pw)(
        edge_index[1], edge_index[0], ep8, hd, hs, zeros_tile)

    out = pl.pallas_call(
        _out_stage,
        out_shape=jax.ShapeDtypeStruct((n, OUT_DIM), f32),
    )(h, parts[0, :n], parts[1, :n], x[:, :OUT_DIM], rpad,
      obj_w1[:HIDDEN], obj_w1[HIDDEN:], obj_b1.reshape(1, HIDDEN),
      obj_w2, obj_b2.reshape(1, HIDDEN), de_w1, de_w2, latent_norm.reshape(1, 1))
    return out


# speedup vs baseline: 2.8464x; 1.0168x over previous
import functools

import jax
import jax.numpy as jnp
from jax import lax
from jax.experimental import pallas as pl
from jax.experimental.pallas import tpu as pltpu
from jax.experimental.pallas import tpu_sc as plsc

N_NODES = 10000
HIDDEN = 40
OUT_DIM = 8
W = 48
L = 16
NC = 2
NS = 16
NW = NC * NS
CHUNK = 128
ROWS_PT = 632
ACC_ROWS = NS * ROWS_PT
ALPHA = 0.5
ALPHA_FCNN = 0.5
NSLOT = 4


def _node_stage(x_ref, w1_ref, w2_ref, wd_ref, ws_ref, h_ref, hd_ref, hs_ref):
    h1 = jnp.maximum(jnp.dot(x_ref[...], w1_ref[...], preferred_element_type=jnp.float32), 0.0)
    h = jnp.dot(h1, w2_ref[...], preferred_element_type=jnp.float32)
    h_ref[...] = h
    hd_ref[...] = jnp.dot(h, wd_ref[...], preferred_element_type=jnp.float32)
    hs_ref[...] = jnp.dot(h, ws_ref[...], preferred_element_type=jnp.float32)


def _edge_stage(ea_ref, w1_ref, w2_ref, wc_ref, brow_ref, ep_ref):
    t = jnp.maximum(jnp.dot(ea_ref[...], w1_ref[...], preferred_element_type=jnp.float32), 0.0)
    e = jnp.dot(t, w2_ref[...], preferred_element_type=jnp.float32)
    ep_ref[...] = jnp.dot(e, wc_ref[...], preferred_element_type=jnp.float32) + brow_ref[...]


def _out_stage(h_ref, p0_ref, p1_ref, xfc_ref, rpad_ref, o1h_ref, o1a_ref,
               ob1_ref, ow2_ref, ob2_ref, dw1_ref, dw2_ref, ln_ref, out_ref):
    p = p0_ref[...] + p1_ref[...]
    aggr = jnp.dot(p, rpad_ref[...], preferred_element_type=jnp.float32)
    h = h_ref[...]
    t = jnp.maximum(
        jnp.dot(h, o1h_ref[...], preferred_element_type=jnp.float32)
        + jnp.dot(aggr, o1a_ref[...], preferred_element_type=jnp.float32)
        + ob1_ref[...], 0.0)
    dx = jnp.dot(t, ow2_ref[...], preferred_element_type=jnp.float32) + ob2_ref[...]
    h2 = ALPHA * h + (1.0 - ALPHA) * dx
    d2 = jnp.dot(jnp.maximum(jnp.dot(h2, dw1_ref[...], preferred_element_type=jnp.float32), 0.0),
                 dw2_ref[...], preferred_element_type=jnp.float32)
    out_ref[...] = (ALPHA_FCNN * xfc_ref[...] + (1.0 - ALPHA_FCNN) * d2) * ln_ref[...]


def _make_sc_edge(n_edges, n_slots_pw):
    mesh = plsc.VectorSubcoreMesh(
        core_axis_name="c", subcore_axis_name="s", num_cores=NC, num_subcores=NS)

    @functools.partial(
        pl.kernel,
        mesh=mesh,
        compiler_params=pltpu.CompilerParams(use_tc_tiling_on_sc=False),
        out_type=jax.ShapeDtypeStruct((NC, ACC_ROWS, W), jnp.float32),
        scratch_types=(
            [pltpu.VMEM((CHUNK,), jnp.int32)] * NSLOT
            + [pltpu.VMEM((CHUNK,), jnp.int32)] * NSLOT
            + [pltpu.VMEM((CHUNK, W), jnp.float32)] * NSLOT
            + [pltpu.VMEM((CHUNK, W), jnp.float32)] * NSLOT
            + [pltpu.VMEM((CHUNK // 8, 8 * W), jnp.float32)] * NSLOT
            + [pltpu.VMEM((CHUNK, W), jnp.float32)] * NSLOT
            + [pltpu.VMEM_SHARED((ACC_ROWS, W), jnp.float32)]
            + [pltpu.SemaphoreType.DMA] * (3 * NSLOT)
        ),
    )
    def sc_edge(dst_hbm, src_hbm, ep8_hbm, hd_hbm, hs_hbm, zero_hbm, out_hbm,
                *bufs):
        dix = bufs[0:NSLOT]
        six = bufs[NSLOT:2 * NSLOT]
        av = bufs[2 * NSLOT:3 * NSLOT]
        bv = bufs[3 * NSLOT:4 * NSLOT]
        cv8 = bufs[4 * NSLOT:5 * NSLOT]
        cv = bufs[5 * NSLOT:6 * NSLOT]
        acc = bufs[6 * NSLOT]
        sa = bufs[6 * NSLOT + 1:6 * NSLOT + 1 + NSLOT]
        sb = bufs[6 * NSLOT + 1 + NSLOT:6 * NSLOT + 1 + 2 * NSLOT]
        se = bufs[6 * NSLOT + 1 + 2 * NSLOT:6 * NSLOT + 1 + 3 * NSLOT]
        cid = lax.axis_index("c")
        sid = lax.axis_index("s")
        pltpu.sync_copy(zero_hbm, acc.at[pl.ds(sid * ROWS_PT, ROWS_PT)])
        plsc.subcore_barrier()
        base = (cid * NS + sid) * (n_slots_pw * CHUNK)
        last = n_edges - CHUNK

        def issue(g, b):
            eb = jnp.minimum(base + g * CHUNK, last)
            pltpu.sync_copy(dst_hbm.at[pl.ds(eb, CHUNK)], dix[b])
            pltpu.sync_copy(src_hbm.at[pl.ds(eb, CHUNK)], six[b])
            cpe = pltpu.async_copy(ep8_hbm.at[pl.ds(eb // 8, CHUNK // 8)], cv8[b], se[b])
            cpa = pltpu.async_copy(hd_hbm.at[dix[b]], av[b], sa[b])
            cpb = pltpu.async_copy(hs_hbm.at[six[b]], bv[b], sb[b])
            return (cpe, cpa, cpb)

        def drain(g, b, handles):
            gate = jnp.where(base + g * CHUNK <= last, 1.0, 0.0).astype(jnp.float32)
            for hnd in handles:
                hnd.wait()

            def inner(r, c2):
                for k in range(8):
                    i = r * 8 + k
                    for j in range(W // L):
                        cv[b][i, pl.ds(j * L, L)] = jnp.maximum(
                            av[b][i, pl.ds(j * L, L)] + bv[b][i, pl.ds(j * L, L)]
                            + cv8[b][r, pl.ds(k * W + j * L, L)], 0.0) * gate
                return c2

            lax.fori_loop(0, CHUNK // 8, inner, 0)
            pltpu.sync_copy(cv[b], acc.at[dix[b]], add=True)

        def body(k, carry):
            g0 = NSLOT * k
            handles = [issue(g0 + b, b) for b in range(NSLOT)]
            for b in range(NSLOT):
                drain(g0 + b, b, handles[b])
            return carry

        lax.fori_loop(0, n_slots_pw // NSLOT, body, 0)
        plsc.subcore_barrier()
        pltpu.sync_copy(acc.at[pl.ds(sid * ROWS_PT, ROWS_PT)],
                        out_hbm.at[cid, pl.ds(sid * ROWS_PT, ROWS_PT)])

    return sc_edge


@jax.jit
def kernel(x, edge_index, edge_attr, ne_w1, ne_w2, ee_w1, ee_w2, rel_w1,
           rel_b1, rel_w2, rel_b2, obj_w1, obj_b1, obj_w2, obj_b2, de_w1,
           de_w2, latent_norm):
    f32 = jnp.float32
    n = x.shape[0]
    e_cnt = edge_attr.shape[0]
    assert e_cnt % CHUNK == 0

    def pad48(w):
        return jnp.concatenate([w, jnp.zeros((w.shape[0], W - HIDDEN), w.dtype)], axis=1)

    wd = pad48(rel_w1[0:HIDDEN])
    ws = pad48(rel_w1[HIDDEN:2 * HIDDEN])
    wc = pad48(rel_w1[2 * HIDDEN:3 * HIDDEN])
    brow = jnp.concatenate(
        [rel_b1, jnp.ones((1,), f32), jnp.zeros((W - HIDDEN - 1,), f32)]).reshape(1, W)

    def blockdiag8(w):
        a, b = w.shape
        out = jnp.zeros((8 * a, 8 * b), w.dtype)
        for k in range(8):
            out = lax.dynamic_update_slice(out, w, (k * a, k * b))
        return out

    w1_blk = blockdiag8(ee_w1)
    w2_blk = blockdiag8(ee_w2)
    wc_blk = blockdiag8(wc)
    brow_blk = jnp.tile(brow, (1, 8))
    rpad = jnp.concatenate(
        [rel_w2, rel_b2.reshape(1, HIDDEN), jnp.zeros((W - HIDDEN - 1, HIDDEN), f32)], axis=0)

    h, hd, hs = pl.pallas_call(
        _node_stage,
        out_shape=[
            jax.ShapeDtypeStruct((n, HIDDEN), f32),
            jax.ShapeDtypeStruct((n, W), f32),
            jax.ShapeDtypeStruct((n, W), f32),
        ],
    )(x, ne_w1, ne_w2, wd, ws)

    e8 = e_cnt // 8
    ea8 = edge_attr.reshape(e8, 8 * edge_attr.shape[1])
    eb8 = 2000
    ep8 = pl.pallas_call(
        _edge_stage,
        grid=(e8 // eb8,),
        in_specs=[
            pl.BlockSpec((eb8, ea8.shape[1]), lambda i: (i, 0)),
            pl.BlockSpec(w1_blk.shape, lambda i: (0, 0)),
            pl.BlockSpec(w2_blk.shape, lambda i: (0, 0)),
            pl.BlockSpec(wc_blk.shape, lambda i: (0, 0)),
            pl.BlockSpec((1, 8 * W), lambda i: (0, 0)),
        ],
        out_specs=pl.BlockSpec((eb8, 8 * W), lambda i: (i, 0)),
        out_shape=jax.ShapeDtypeStruct((e8, 8 * W), f32),
    )(ea8, w1_blk, w2_blk, wc_blk, brow_blk)

    n_chunks = e_cnt // CHUNK
    n_chunks_pw = (n_chunks + NW - 1) // NW
    n_chunks_pw = ((n_chunks_pw + NSLOT - 1) // NSLOT) * NSLOT
    zeros_tile = jnp.zeros((ROWS_PT, W), f32)

    parts = _make_sc_edge(e_cnt, n_chunks_pw)(
        edge_index[1], edge_index[0], ep8, hd, hs, zeros_tile)

    out = pl.pallas_call(
        _out_stage,
        out_shape=jax.ShapeDtypeStruct((n, OUT_DIM), f32),
    )(h, parts[0, :n], parts[1, :n], x[:, :OUT_DIM], rpad,
      obj_w1[:HIDDEN], obj_w1[HIDDEN:], obj_b1.reshape(1, HIDDEN),
      obj_w2, obj_b2.reshape(1, HIDDEN), de_w1, de_w2, latent_norm.reshape(1, 1))
    return out


# speedup vs baseline: 3.0898x; 1.0855x over previous
import functools

import jax
import jax.numpy as jnp
from jax import lax
from jax.experimental import pallas as pl
from jax.experimental.pallas import tpu as pltpu
from jax.experimental.pallas import tpu_sc as plsc

N_NODES = 10000
HIDDEN = 40
OUT_DIM = 8
W = 48
L = 16
NC = 2
NS = 16
NW = NC * NS
CHUNK = 128
ROWS_PT = 632
ACC_ROWS = NS * ROWS_PT
ALPHA = 0.5
ALPHA_FCNN = 0.5
NSLOT = 4


def _node_stage(x_ref, w1_ref, w2_ref, wd_ref, ws_ref, h_ref, hd_ref, hs_ref):
    h1 = jnp.maximum(jnp.dot(x_ref[...], w1_ref[...], preferred_element_type=jnp.float32), 0.0)
    h = jnp.dot(h1, w2_ref[...], preferred_element_type=jnp.float32)
    h_ref[...] = h
    hd_ref[...] = jnp.dot(h, wd_ref[...], preferred_element_type=jnp.float32)
    hs_ref[...] = jnp.dot(h, ws_ref[...], preferred_element_type=jnp.float32)


def _edge_stage(ea_ref, w1_ref, w2_ref, wc_ref, brow_ref, ep_ref):
    t = jnp.maximum(jnp.dot(ea_ref[...], w1_ref[...], preferred_element_type=jnp.float32), 0.0)
    e = jnp.dot(t, w2_ref[...], preferred_element_type=jnp.float32)
    ep_ref[...] = jnp.dot(e, wc_ref[...], preferred_element_type=jnp.float32) + brow_ref[...]


def _out_stage(h_ref, p0_ref, p1_ref, xfc_ref, rpad_ref, o1h_ref, o1a_ref,
               ob1_ref, ow2_ref, ob2_ref, dw1_ref, dw2_ref, ln_ref, out_ref):
    p = p0_ref[...] + p1_ref[...]
    aggr = jnp.dot(p, rpad_ref[...], preferred_element_type=jnp.float32)
    h = h_ref[...]
    t = jnp.maximum(
        jnp.dot(h, o1h_ref[...], preferred_element_type=jnp.float32)
        + jnp.dot(aggr, o1a_ref[...], preferred_element_type=jnp.float32)
        + ob1_ref[...], 0.0)
    dx = jnp.dot(t, ow2_ref[...], preferred_element_type=jnp.float32) + ob2_ref[...]
    h2 = ALPHA * h + (1.0 - ALPHA) * dx
    d2 = jnp.dot(jnp.maximum(jnp.dot(h2, dw1_ref[...], preferred_element_type=jnp.float32), 0.0),
                 dw2_ref[...], preferred_element_type=jnp.float32)
    out_ref[...] = (ALPHA_FCNN * xfc_ref[...] + (1.0 - ALPHA_FCNN) * d2) * ln_ref[...]


def _make_sc_edge(n_edges, n_slots_pw):
    mesh = plsc.VectorSubcoreMesh(
        core_axis_name="c", subcore_axis_name="s", num_cores=NC, num_subcores=NS)

    @functools.partial(
        pl.kernel,
        mesh=mesh,
        compiler_params=pltpu.CompilerParams(use_tc_tiling_on_sc=False),
        out_type=jax.ShapeDtypeStruct((NC, ACC_ROWS, W), jnp.float32),
        scratch_types=(
            [pltpu.VMEM((NSLOT, CHUNK), jnp.int32)] * 2
            + [pltpu.VMEM((NSLOT, CHUNK), jnp.int32)] * 2
            + [pltpu.VMEM((CHUNK, W), jnp.float32)] * NSLOT
            + [pltpu.VMEM((CHUNK, W), jnp.float32)] * NSLOT
            + [pltpu.VMEM((CHUNK // 8, 8 * W), jnp.float32)] * NSLOT
            + [pltpu.VMEM((CHUNK, W), jnp.float32)] * NSLOT
            + [pltpu.VMEM_SHARED((ACC_ROWS, W), jnp.float32)]
            + [pltpu.SemaphoreType.DMA] * (3 * NSLOT)
            + [pltpu.SemaphoreType.DMA] * 2
        ),
    )
    def sc_edge(dst_hbm, src_hbm, ep8_hbm, hd_hbm, hs_hbm, zero_hbm, out_hbm,
                *bufs):
        dixg = bufs[0:2]
        sixg = bufs[2:4]
        av = bufs[4:4 + NSLOT]
        bv = bufs[4 + NSLOT:4 + 2 * NSLOT]
        cv8 = bufs[4 + 2 * NSLOT:4 + 3 * NSLOT]
        cv = bufs[4 + 3 * NSLOT:4 + 4 * NSLOT]
        acc = bufs[4 + 4 * NSLOT]
        sa = bufs[5 + 4 * NSLOT:5 + 5 * NSLOT]
        sb = bufs[5 + 5 * NSLOT:5 + 6 * NSLOT]
        se = bufs[5 + 6 * NSLOT:5 + 7 * NSLOT]
        si = bufs[5 + 7 * NSLOT:7 + 7 * NSLOT]
        cid = lax.axis_index("c")
        sid = lax.axis_index("s")
        pltpu.sync_copy(zero_hbm, acc.at[pl.ds(sid * ROWS_PT, ROWS_PT)])
        plsc.subcore_barrier()
        n_ch = n_edges // CHUNK
        base_c = (cid * NS + sid) * n_slots_pw
        last_g = n_ch - NSLOT

        def gstart(grp):
            return jnp.minimum(base_c + grp * NSLOT, last_g)

        def issue_idx(grp, p):
            gs = gstart(grp)
            pltpu.async_copy(dst_hbm.at[pl.ds(gs, NSLOT)], dixg[p], si[p])
            pltpu.async_copy(src_hbm.at[pl.ds(gs, NSLOT)], sixg[p], si[p])

        def wait_idx(p):
            pltpu.make_async_copy(dst_hbm.at[pl.ds(0, NSLOT)], dixg[p], si[p]).wait()
            pltpu.make_async_copy(src_hbm.at[pl.ds(0, NSLOT)], sixg[p], si[p]).wait()

        def issue(grp, b, p):
            rc = gstart(grp) + b
            cpe = pltpu.async_copy(ep8_hbm.at[pl.ds(rc * (CHUNK // 8), CHUNK // 8)],
                                   cv8[b], se[b])
            cpa = pltpu.async_copy(hd_hbm.at[dixg[p].at[b]], av[b], sa[b])
            cpb = pltpu.async_copy(hs_hbm.at[sixg[p].at[b]], bv[b], sb[b])
            return (cpe, cpa, cpb)

        def drain(grp, b, p, handles):
            gate = jnp.where(base_c + grp * NSLOT + b <= n_ch - 1, 1.0, 0.0
                             ).astype(jnp.float32)
            for hnd in handles:
                hnd.wait()

            def inner(r, c2):
                for k in range(8):
                    i = r * 8 + k
                    for j in range(W // L):
                        cv[b][i, pl.ds(j * L, L)] = jnp.maximum(
                            av[b][i, pl.ds(j * L, L)] + bv[b][i, pl.ds(j * L, L)]
                            + cv8[b][r, pl.ds(k * W + j * L, L)], 0.0) * gate
                return c2

            lax.fori_loop(0, CHUNK // 8, inner, 0)
            pltpu.sync_copy(cv[b], acc.at[dixg[p].at[b]], add=True)

        def half(grp, p, nxt):
            wait_idx(p)
            issue_idx(nxt, 1 - p)
            handles = [issue(grp, b, p) for b in range(NSLOT)]
            for b in range(NSLOT):
                drain(grp, b, p, handles[b])

        issue_idx(0, 0)

        def body(k, carry):
            half(2 * k, 0, 2 * k + 1)
            half(2 * k + 1, 1, 2 * k + 2)
            return carry

        n_groups = n_slots_pw // NSLOT
        lax.fori_loop(0, n_groups // 2, body, 0)
        wait_idx(0)
        plsc.subcore_barrier()
        pltpu.sync_copy(acc.at[pl.ds(sid * ROWS_PT, ROWS_PT)],
                        out_hbm.at[cid, pl.ds(sid * ROWS_PT, ROWS_PT)])

    return sc_edge


@jax.jit
def kernel(x, edge_index, edge_attr, ne_w1, ne_w2, ee_w1, ee_w2, rel_w1,
           rel_b1, rel_w2, rel_b2, obj_w1, obj_b1, obj_w2, obj_b2, de_w1,
           de_w2, latent_norm):
    f32 = jnp.float32
    n = x.shape[0]
    e_cnt = edge_attr.shape[0]
    assert e_cnt % CHUNK == 0

    def pad48(w):
        return jnp.concatenate([w, jnp.zeros((w.shape[0], W - HIDDEN), w.dtype)], axis=1)

    wd = pad48(rel_w1[0:HIDDEN])
    ws = pad48(rel_w1[HIDDEN:2 * HIDDEN])
    wc = pad48(rel_w1[2 * HIDDEN:3 * HIDDEN])
    brow = jnp.concatenate(
        [rel_b1, jnp.ones((1,), f32), jnp.zeros((W - HIDDEN - 1,), f32)]).reshape(1, W)

    def blockdiag8(w):
        a, b = w.shape
        out = jnp.zeros((8 * a, 8 * b), w.dtype)
        for k in range(8):
            out = lax.dynamic_update_slice(out, w, (k * a, k * b))
        return out

    w1_blk = blockdiag8(ee_w1)
    w2_blk = blockdiag8(ee_w2)
    wc_blk = blockdiag8(wc)
    brow_blk = jnp.tile(brow, (1, 8))
    rpad = jnp.concatenate(
        [rel_w2, rel_b2.reshape(1, HIDDEN), jnp.zeros((W - HIDDEN - 1, HIDDEN), f32)], axis=0)

    h, hd, hs = pl.pallas_call(
        _node_stage,
        out_shape=[
            jax.ShapeDtypeStruct((n, HIDDEN), f32),
            jax.ShapeDtypeStruct((n, W), f32),
            jax.ShapeDtypeStruct((n, W), f32),
        ],
    )(x, ne_w1, ne_w2, wd, ws)

    e8 = e_cnt // 8
    ea8 = edge_attr.reshape(e8, 8 * edge_attr.shape[1])
    eb8 = 2000
    ep8 = pl.pallas_call(
        _edge_stage,
        grid=(e8 // eb8,),
        in_specs=[
            pl.BlockSpec((eb8, ea8.shape[1]), lambda i: (i, 0)),
            pl.BlockSpec(w1_blk.shape, lambda i: (0, 0)),
            pl.BlockSpec(w2_blk.shape, lambda i: (0, 0)),
            pl.BlockSpec(wc_blk.shape, lambda i: (0, 0)),
            pl.BlockSpec((1, 8 * W), lambda i: (0, 0)),
        ],
        out_specs=pl.BlockSpec((eb8, 8 * W), lambda i: (i, 0)),
        out_shape=jax.ShapeDtypeStruct((e8, 8 * W), f32),
    )(ea8, w1_blk, w2_blk, wc_blk, brow_blk)

    n_chunks = e_cnt // CHUNK
    n_chunks_pw = (n_chunks + NW - 1) // NW
    n_chunks_pw = ((n_chunks_pw + 2 * NSLOT - 1) // (2 * NSLOT)) * 2 * NSLOT
    zeros_tile = jnp.zeros((ROWS_PT, W), f32)

    dst2 = edge_index[1].reshape(e_cnt // CHUNK, CHUNK)
    src2 = edge_index[0].reshape(e_cnt // CHUNK, CHUNK)
    parts = _make_sc_edge(e_cnt, n_chunks_pw)(
        dst2, src2, ep8, hd, hs, zeros_tile)

    out = pl.pallas_call(
        _out_stage,
        out_shape=jax.ShapeDtypeStruct((n, OUT_DIM), f32),
    )(h, parts[0, :n], parts[1, :n], x[:, :OUT_DIM], rpad,
      obj_w1[:HIDDEN], obj_w1[HIDDEN:], obj_b1.reshape(1, HIDDEN),
      obj_w2, obj_b2.reshape(1, HIDDEN), de_w1, de_w2, latent_norm.reshape(1, 1))
    return out
